# baseline (speedup 1.0000x reference)
import jax
import jax.numpy as jnp
from jax import lax
from jax.experimental import pallas as pl
from jax.experimental.pallas import tpu as pltpu
from jax.experimental.pallas import tpu_sc as plsc

_QR_RATIO = 64
_EMB_DIM = 64
_LANES = 16
_NC = 2
_NS = 16
_NW = _NC * _NS
_QROWS = 15627

_B = 16384 * 26
_PW = _B // _NW
_C = 416
_NCH = _PW // _C


def _body(x_hbm, embq_hbm, embr_hbm, out_hbm,
          spq, spr, idxall, qi0, qi1, ri0, ri1, rq0, rq1,
          semi, semq0, semq1, semr0, semr1, semo0, semo1, semo20, semo21):
    wid = lax.axis_index("s") * _NC + lax.axis_index("c")
    base_w = wid * _PW
    qi, ri, rq = [qi0, qi1], [ri0, ri1], [rq0, rq1]
    semq, semr, semo = [semq0, semq1], [semr0, semr1], [semo0, semo1]
    semo2 = [semo20, semo21]

    _H = _C // 2

    def out_copy(ch, b):
        return pltpu.make_async_copy(
            rq[b].at[pl.ds(0, _H)],
            out_hbm.at[pl.ds(base_w + ch * _C, _H)], semo[b])

    def out_copy2(ch, b):
        return pltpu.make_async_copy(
            rq[b].at[pl.ds(_H, _H)],
            out_hbm.at[pl.ds(base_w + ch * _C + _H, _H)], semo2[b])

    def q_copy(b):
        return pltpu.make_async_copy(spq.at[qi[b].at[0]], rq[b], semq[b])

    def r_copy(b):
        return pltpu.make_async_copy(spr.at[ri[b].at[0]], rq[b], semr[b])

    def compute_qr(ch, b):
        for i in range(_C // _LANES):
            v = idxall[pl.ds(ch * _C + i * _LANES, _LANES)]
            o = pl.ds(i * _LANES, _LANES)
            qi[b][0, o] = v >> 6
            ri[b][0, o] = v & (_QR_RATIO - 1)

    idx_cp = pltpu.make_async_copy(
        x_hbm.at[pl.ds(base_w, _PW)], idxall, semi)
    idx_cp.start()

    @pl.when(lax.axis_index("s") == 0)
    def _stage():
        pltpu.sync_copy(embq_hbm, spq)
        pltpu.sync_copy(embr_hbm, spr)
    plsc.subcore_barrier()

    idx_cp.wait()
    compute_qr(0, 0)
    q_copy(0).start()

    def iter_body(p, carry):
        for b in (0, 1):
            ch = p * 2 + b
            nb = 1 - b

            @pl.when(ch + 1 < _NCH)
            def _prep():
                compute_qr(ch + 1, nb)

                @pl.when(ch >= 1)
                def _wo():
                    out_copy(ch - 1, nb).wait()
                    out_copy2(ch - 1, nb).wait()
                q_copy(nb).start()

            q_copy(b).wait()
            r_copy(b).start(add=True)
            r_copy(b).wait()
            out_copy(ch, b).start()
            out_copy2(ch, b).start()
        return carry

    lax.fori_loop(0, _NCH // 2, iter_body, 0)
    out_copy(_NCH - 2, 0).wait()
    out_copy2(_NCH - 2, 0).wait()
    out_copy(_NCH - 1, 1).wait()
    out_copy2(_NCH - 1, 1).wait()


@jax.jit
def _qr_embed(x_flat, embedding_q, embedding_r):
    mesh = plsc.VectorSubcoreMesh(
        core_axis_name="c", subcore_axis_name="s",
        num_cores=_NC, num_subcores=_NS)
    return pl.kernel(
        _body,
        out_type=jax.ShapeDtypeStruct((_B, _EMB_DIM), jnp.float32),
        mesh=mesh,
        scratch_types=[
            pltpu.VMEM_SHARED((_QROWS, _EMB_DIM), jnp.float32),
            pltpu.VMEM_SHARED((_QR_RATIO, _EMB_DIM), jnp.float32),
            pltpu.VMEM((_PW,), jnp.int32),
            pltpu.VMEM((1, _C), jnp.int32),
            pltpu.VMEM((1, _C), jnp.int32),
            pltpu.VMEM((1, _C), jnp.int32),
            pltpu.VMEM((1, _C), jnp.int32),
            pltpu.VMEM((_C, _EMB_DIM), jnp.float32),
            pltpu.VMEM((_C, _EMB_DIM), jnp.float32),
            pltpu.SemaphoreType.DMA,
            pltpu.SemaphoreType.DMA,
            pltpu.SemaphoreType.DMA,
            pltpu.SemaphoreType.DMA,
            pltpu.SemaphoreType.DMA,
            pltpu.SemaphoreType.DMA,
            pltpu.SemaphoreType.DMA,
            pltpu.SemaphoreType.DMA,
            pltpu.SemaphoreType.DMA,
        ],
        compiler_params=pltpu.CompilerParams(
            use_tc_tiling_on_sc=False, needs_layout_passes=False),
    )(x_flat, embedding_q, embedding_r)


def kernel(x, embedding_q, embedding_r):
    b, f = x.shape
    x_flat = x.reshape(-1).astype(jnp.int32)
    out = _qr_embed(x_flat, embedding_q, embedding_r)
    return out.reshape(b, f, _EMB_DIM)


# speedup vs baseline: 1.0036x; 1.0036x over previous
import jax
import jax.numpy as jnp
from jax import lax
from jax.experimental import pallas as pl
from jax.experimental.pallas import tpu as pltpu
from jax.experimental.pallas import tpu_sc as plsc

_QR_RATIO = 64
_EMB_DIM = 64
_LANES = 16
_NC = 2
_NS = 16
_NW = _NC * _NS
_QROWS = 15627

_B = 16384 * 26
_PW = _B // _NW
_C = 416
_NCH = _PW // _C


def _body(x_hbm, embq_hbm, embr_hbm, out_hbm,
          spq, spr, idxall, qi0, qi1, ri0, ri1, rq0, rq1,
          semi, semq0, semq1, semr0, semr1, semo0, semo1):
    wid = lax.axis_index("s") * _NC + lax.axis_index("c")
    base_w = wid * _PW
    qi, ri, rq = [qi0, qi1], [ri0, ri1], [rq0, rq1]
    semq, semr, semo = [semq0, semq1], [semr0, semr1], [semo0, semo1]

    def out_copy(ch, b):
        return pltpu.make_async_copy(
            rq[b], out_hbm.at[pl.ds(base_w + ch * _C, _C)], semo[b])

    def q_copy(b):
        return pltpu.make_async_copy(spq.at[qi[b].at[0]], rq[b], semq[b])

    def r_copy(b):
        return pltpu.make_async_copy(spr.at[ri[b].at[0]], rq[b], semr[b])

    def compute_qr(ch, b):
        for i in range(_C // _LANES):
            v = idxall[pl.ds(ch * _C + i * _LANES, _LANES)]
            o = pl.ds(i * _LANES, _LANES)
            qi[b][0, o] = v >> 6
            ri[b][0, o] = v & (_QR_RATIO - 1)

    idx_cp = pltpu.make_async_copy(
        x_hbm.at[pl.ds(base_w, _PW)], idxall, semi)
    idx_cp.start()

    @pl.when(lax.axis_index("s") == 0)
    def _stage():
        pltpu.sync_copy(embq_hbm, spq)
        pltpu.sync_copy(embr_hbm, spr)
    plsc.subcore_barrier()

    idx_cp.wait()
    compute_qr(0, 0)
    q_copy(0).start()

    def iter_body(p, carry):
        for b in (0, 1):
            ch = p * 2 + b
            nb = 1 - b

            @pl.when(ch + 1 < _NCH)
            def _prep():
                compute_qr(ch + 1, nb)

                @pl.when(ch >= 1)
                def _wo():
                    out_copy(ch - 1, nb).wait()
                q_copy(nb).start()

            q_copy(b).wait()
            r_copy(b).start(add=True)
            r_copy(b).wait()
            out_copy(ch, b).start()
        return carry

    lax.fori_loop(0, _NCH // 2, iter_body, 0)
    out_copy(_NCH - 2, 0).wait()
    out_copy(_NCH - 1, 1).wait()


@jax.jit
def _qr_embed(x_flat, embedding_q, embedding_r):
    mesh = plsc.VectorSubcoreMesh(
        core_axis_name="c", subcore_axis_name="s",
        num_cores=_NC, num_subcores=_NS)
    return pl.kernel(
        _body,
        out_type=jax.ShapeDtypeStruct((_B, _EMB_DIM), jnp.float32),
        mesh=mesh,
        scratch_types=[
            pltpu.VMEM_SHARED((_QROWS, _EMB_DIM), jnp.float32),
            pltpu.VMEM_SHARED((_QR_RATIO, _EMB_DIM), jnp.float32),
            pltpu.VMEM((_PW,), jnp.int32),
            pltpu.VMEM((1, _C), jnp.int32),
            pltpu.VMEM((1, _C), jnp.int32),
            pltpu.VMEM((1, _C), jnp.int32),
            pltpu.VMEM((1, _C), jnp.int32),
            pltpu.VMEM((_C, _EMB_DIM), jnp.float32),
            pltpu.VMEM((_C, _EMB_DIM), jnp.float32),
            pltpu.SemaphoreType.DMA,
            pltpu.SemaphoreType.DMA,
            pltpu.SemaphoreType.DMA,
            pltpu.SemaphoreType.DMA,
            pltpu.SemaphoreType.DMA,
            pltpu.SemaphoreType.DMA,
            pltpu.SemaphoreType.DMA,
        ],
        compiler_params=pltpu.CompilerParams(
            use_tc_tiling_on_sc=False, needs_layout_passes=False),
    )(x_flat, embedding_q, embedding_r)


def kernel(x, embedding_q, embedding_r):
    b, f = x.shape
    x_flat = x.reshape(-1).astype(jnp.int32)
    out = _qr_embed(x_flat, embedding_q, embedding_r)
    return out.reshape(b, f, _EMB_DIM)
